# baseline (speedup 1.0000x reference)
import functools

import jax
import jax.numpy as jnp
from jax import lax
from jax.experimental import pallas as pl
from jax.experimental.pallas import tpu as pltpu
from jax.experimental.pallas import tpu_sc as plsc

N = 50000
E = 800000
D = 64
HALF = 32
NCORE = 2
NSUB = 16
CHUNK = 128
NCHUNK = 392
EPS = CHUNK * NCHUNK
E_PAD = EPS * NSUB
STRIPE = 3128
NP = STRIPE * NSUB


BLK = 4
NBLK = NCHUNK // BLK
RB = 4


def _spmm_sc(srcp, dstp, wp, xflat, zrows):
    mesh = plsc.VectorSubcoreMesh(
        core_axis_name="c", subcore_axis_name="s", num_cores=NCORE, num_subcores=NSUB
    )

    @functools.partial(
        pl.kernel,
        out_type=jax.ShapeDtypeStruct((NCORE, NP, HALF), jnp.float32),
        mesh=mesh,
        scratch_types=[
            pltpu.VMEM((2, BLK, CHUNK), jnp.int32),
            pltpu.VMEM((2, BLK, CHUNK), jnp.int32),
            pltpu.VMEM((2, BLK, CHUNK), jnp.float32),
            pltpu.VMEM((RB, CHUNK), jnp.int32),
            pltpu.VMEM((RB, CHUNK, HALF), jnp.float32),
            pltpu.VMEM_SHARED((NP, HALF), jnp.float32),
            [pltpu.SemaphoreType.DMA] * RB,
            [pltpu.SemaphoreType.DMA] * RB,
            [pltpu.SemaphoreType.DMA] * 2,
            pltpu.SemaphoreType.DMA,
        ],
        compiler_params=pltpu.CompilerParams(use_tc_tiling_on_sc=False),
    )
    def k(src_hbm, dst_hbm, w_hbm, x_hbm, z_hbm, y_hbm,
          src2, dst2, w2, dstr, rows, acc, sem_g, sem_s, sem_i, sem_z):
        c = lax.axis_index("c")
        s = lax.axis_index("s")
        srcc = src_hbm
        xc = x_hbm.at[c]
        ebase = s * NCHUNK

        def idx_row(blk):
            return ebase + blk * BLK

        def load_idx(blk, ib, sem):
            r = idx_row(blk)
            pltpu.async_copy(srcc.at[pl.ds(r, BLK)], src2.at[ib], sem)
            pltpu.async_copy(dst_hbm.at[pl.ds(r, BLK)], dst2.at[ib], sem)
            pltpu.async_copy(w_hbm.at[pl.ds(r, BLK)], w2.at[ib], sem)

        def wait_idx(blk, ib, sem):
            r = idx_row(blk)
            pltpu.make_async_copy(srcc.at[pl.ds(r, BLK)], src2.at[ib], sem).wait()
            pltpu.make_async_copy(dst_hbm.at[pl.ds(r, BLK)], dst2.at[ib], sem).wait()
            pltpu.make_async_copy(w_hbm.at[pl.ds(r, BLK)], w2.at[ib], sem).wait()

        def bias_and_gather(ib, row, rb):
            pltpu.async_copy(xc.at[src2.at[ib, row]], rows.at[rb], sem_g[rb])

        def wait_gather(ib, row, rb):
            pltpu.make_async_copy(
                xc.at[src2.at[ib, row]], rows.at[rb], sem_g[rb]
            ).wait()

        def issue_scatter(ib, row, rb):
            for i in range(CHUNK // 16):
                sl = pl.ds(i * 16, 16)
                dstr[rb, sl] = dst2[ib, row, sl]
            pltpu.async_copy(rows.at[rb], acc.at[dstr.at[rb]], sem_s[rb], add=True)

        def wait_scatter(rb):
            pltpu.make_async_copy(
                rows.at[rb], acc.at[dstr.at[rb]], sem_s[rb]
            ).wait()

        def scale(ib, row, rb):
            @pl.loop(0, CHUNK, step=16)
            def _(g):
                w16 = w2[ib, row, pl.ds(g, 16)]
                for kk in range(16):
                    r = g + kk
                    wr = w16[kk]
                    rows[rb, r, pl.ds(0, 16)] = rows[rb, r, pl.ds(0, 16)] * wr
                    rows[rb, r, pl.ds(16, 16)] = rows[rb, r, pl.ds(16, 16)] * wr

        pltpu.async_copy(
            z_hbm.at[pl.ds(s * STRIPE, STRIPE)],
            acc.at[pl.ds(s * STRIPE, STRIPE)],
            sem_z,
        )
        load_idx(0, 0, sem_i[0])
        wait_idx(0, 0, sem_i[0])
        load_idx(1, 1, sem_i[1])
        for kk in range(RB - 1):
            bias_and_gather(0, kk, kk)
        pltpu.make_async_copy(
            z_hbm.at[pl.ds(s * STRIPE, STRIPE)],
            acc.at[pl.ds(s * STRIPE, STRIPE)],
            sem_z,
        ).wait()
        plsc.subcore_barrier()

        @pl.loop(0, NBLK // 2)
        def _(g):
            for slot in range(2):
                ib = slot
                for kk in range(BLK):
                    wait_gather(ib, kk, kk)
                    scale(ib, kk, kk)
                    if kk == 0:
                        tib, trow = ib, 3
                    else:
                        tib, trow = 1 - ib, kk - 1
                    last_block = slot == 1 and kk >= 1
                    if kk == 1:
                        def _w(g=g, slot=slot, ib=ib):
                            wait_idx(2 * g + slot + 1, 1 - ib, sem_i[1 - ib])
                        if slot == 0:
                            _w()
                        else:
                            pl.when(g < NBLK // 2 - 1)(_w)
                    def _ws(trow=trow):
                        wait_scatter(trow)
                    if kk == 0 and slot == 0:
                        pl.when(g > 0)(_ws)
                    elif last_block:
                        pl.when(g < NBLK // 2 - 1)(_ws)
                    else:
                        _ws()
                    def _ig(tib=tib, trow=trow):
                        bias_and_gather(tib, trow, trow)
                    if last_block:
                        pl.when(g < NBLK // 2 - 1)(_ig)
                    else:
                        _ig()
                    issue_scatter(ib, kk, kk)
                    if kk == 3:
                        def _li(g=g, slot=slot, ib=ib):
                            load_idx(2 * g + slot + 2, ib, sem_i[ib])
                        pl.when(g < NBLK // 2 - 1)(_li)

        for kk in range(BLK):
            wait_scatter(kk)
        plsc.subcore_barrier()
        pltpu.sync_copy(
            acc.at[pl.ds(s * STRIPE, STRIPE)],
            y_hbm.at[c, pl.ds(s * STRIPE, STRIPE)],
        )

    return k(srcp, dstp, wp, xflat, zrows)


G4 = NP // 4
_DB = G4 // 4
_DG = 4


def _dense_mid_body(y_ref, wa_ref, wb_ref, b_ref, o_ref):
    y0, y1 = y_ref[0], y_ref[1]
    for cc in range(NCORE):
        h = (
            jnp.dot(y0, wa_ref[cc], preferred_element_type=jnp.float32)
            + jnp.dot(y1, wb_ref[cc], preferred_element_type=jnp.float32)
            + b_ref[cc]
        )
        o_ref[cc] = jnp.maximum(h, 0.0)


def _dense_mid_tc(yp, wa, wb, bb):
    return pl.pallas_call(
        _dense_mid_body,
        grid=(_DG,),
        in_specs=[
            pl.BlockSpec((NCORE, _DB, 4 * HALF), lambda i: (0, i, 0)),
            pl.BlockSpec((NCORE, 4 * HALF, 4 * HALF), lambda i: (0, 0, 0)),
            pl.BlockSpec((NCORE, 4 * HALF, 4 * HALF), lambda i: (0, 0, 0)),
            pl.BlockSpec((NCORE, 1, 4 * HALF), lambda i: (0, 0, 0)),
        ],
        out_specs=pl.BlockSpec((NCORE, _DB, 4 * HALF), lambda i: (0, i, 0)),
        out_shape=jax.ShapeDtypeStruct((NCORE, G4, 4 * HALF), jnp.float32),
    )(yp, wa, wb, bb)


def _dense_final_body(y_ref, ca_ref, cb_ref, b_ref, o_ref):
    h = (
        jnp.dot(y_ref[0], ca_ref[...], preferred_element_type=jnp.float32)
        + jnp.dot(y_ref[1], cb_ref[...], preferred_element_type=jnp.float32)
        + b_ref[...]
    )
    o_ref[...] = jnp.maximum(h, 0.0)


def _dense_final_tc(yp, ca, cb, bb):
    return pl.pallas_call(
        _dense_final_body,
        grid=(_DG,),
        in_specs=[
            pl.BlockSpec((NCORE, _DB, 4 * HALF), lambda i: (0, i, 0)),
            pl.BlockSpec((4 * HALF, 4 * D), lambda i: (0, 0)),
            pl.BlockSpec((4 * HALF, 4 * D), lambda i: (0, 0)),
            pl.BlockSpec((1, 4 * D), lambda i: (0, 0)),
        ],
        out_specs=pl.BlockSpec((_DB, 4 * D), lambda i: (i, 0)),
        out_shape=jax.ShapeDtypeStruct((G4, 4 * D), jnp.float32),
    )(yp, ca, cb, bb)


def _blockdiag4(w32):
    z = jnp.zeros((4 * HALF, 4 * HALF), jnp.float32)
    for i in range(4):
        z = z.at[32 * i : 32 * i + 32, 32 * i : 32 * i + 32].set(w32)
    return z


def _blockrow4(w32x64):
    z = jnp.zeros((4 * HALF, 4 * D), jnp.float32)
    for i in range(4):
        z = z.at[32 * i : 32 * i + 32, 64 * i : 64 * i + 64].set(w32x64)
    return z


def kernel(edge_index, edge_weight, emb_weight, W1, b1, W2, b2):
    pad = E_PAD - E
    srcp = jnp.pad(edge_index[1], (0, pad)).reshape(E_PAD // CHUNK, CHUNK)
    dstp = jnp.pad(edge_index[0], (0, pad)).reshape(E_PAD // CHUNK, CHUNK)
    wp = jnp.pad(edge_weight, (0, pad)).reshape(E_PAD // CHUNK, CHUNK)

    e4 = emb_weight.reshape(N // 4, 4, D)
    xb = jnp.stack(
        [e4[:, :, 0:HALF].reshape(N // 4, 4 * HALF),
         e4[:, :, HALF:D].reshape(N // 4, 4 * HALF)]
    )
    xb = jnp.pad(xb, ((0, 0), (0, G4 - N // 4), (0, 0)))
    zrows = jnp.zeros((NP, HALF), jnp.float32)
    wa1 = jnp.stack([_blockdiag4(W1[0:HALF, 0:HALF]), _blockdiag4(W1[0:HALF, HALF:D])])
    wb1 = jnp.stack([_blockdiag4(W1[HALF:D, 0:HALF]), _blockdiag4(W1[HALF:D, HALF:D])])
    bt1 = jnp.stack([jnp.tile(b1[0:HALF], 4).reshape(1, 4 * HALF),
                     jnp.tile(b1[HALF:D], 4).reshape(1, 4 * HALF)])
    ca2 = _blockrow4(W2[0:HALF, :])
    cb2 = _blockrow4(W2[HALF:D, :])
    bt2 = jnp.tile(b2, 4).reshape(1, 4 * D)

    y1 = _spmm_sc(srcp, dstp, wp, xb.reshape(NCORE, NP, HALF), zrows)
    h = _dense_mid_tc(y1.reshape(NCORE, G4, 4 * HALF), wa1, wb1, bt1)
    y2 = _spmm_sc(srcp, dstp, wp, h.reshape(NCORE, NP, HALF), zrows)
    zp = _dense_final_tc(y2.reshape(NCORE, G4, 4 * HALF), ca2, cb2, bt2)
    return zp.reshape(NP, D)[0:N]


# speedup vs baseline: 1.0295x; 1.0295x over previous
import functools

import jax
import jax.numpy as jnp
from jax import lax
from jax.experimental import pallas as pl
from jax.experimental.pallas import tpu as pltpu
from jax.experimental.pallas import tpu_sc as plsc

N = 50000
E = 800000
D = 64
HALF = 32
NCORE = 2
NSUB = 16
CHUNK = 128
NCHUNK = 392
EPS = CHUNK * NCHUNK
E_PAD = EPS * NSUB
STRIPE = 3128
NP = STRIPE * NSUB


BLK = 4
NBLK = NCHUNK // BLK
RB = 4


def _spmm_sc(srcp, dstp, wp, xflat, zrows):
    mesh = plsc.VectorSubcoreMesh(
        core_axis_name="c", subcore_axis_name="s", num_cores=NCORE, num_subcores=NSUB
    )

    @functools.partial(
        pl.kernel,
        out_type=jax.ShapeDtypeStruct((NP, NCORE, HALF), jnp.float32),
        mesh=mesh,
        scratch_types=[
            pltpu.VMEM((2, BLK, CHUNK), jnp.int32),
            pltpu.VMEM((2, BLK, CHUNK), jnp.int32),
            pltpu.VMEM((2, BLK, CHUNK), jnp.float32),
            pltpu.VMEM((RB, CHUNK), jnp.int32),
            pltpu.VMEM((RB, CHUNK, HALF), jnp.float32),
            pltpu.VMEM_SHARED((NP, HALF), jnp.float32),
            [pltpu.SemaphoreType.DMA] * RB,
            [pltpu.SemaphoreType.DMA] * RB,
            [pltpu.SemaphoreType.DMA] * 2,
            pltpu.SemaphoreType.DMA,
        ],
        compiler_params=pltpu.CompilerParams(use_tc_tiling_on_sc=False),
    )
    def k(src_hbm, dst_hbm, w_hbm, x_hbm, z_hbm, y_hbm,
          src2, dst2, w2, dstr, rows, acc, sem_g, sem_s, sem_i, sem_z):
        c = lax.axis_index("c")
        s = lax.axis_index("s")
        srcc = src_hbm
        ebase = s * NCHUNK

        def idx_row(blk):
            return ebase + blk * BLK

        def load_idx(blk, ib, sem):
            r = idx_row(blk)
            pltpu.async_copy(srcc.at[pl.ds(r, BLK)], src2.at[ib], sem)
            pltpu.async_copy(dst_hbm.at[pl.ds(r, BLK)], dst2.at[ib], sem)
            pltpu.async_copy(w_hbm.at[pl.ds(r, BLK)], w2.at[ib], sem)

        def wait_idx(blk, ib, sem):
            r = idx_row(blk)
            pltpu.make_async_copy(srcc.at[pl.ds(r, BLK)], src2.at[ib], sem).wait()
            pltpu.make_async_copy(dst_hbm.at[pl.ds(r, BLK)], dst2.at[ib], sem).wait()
            pltpu.make_async_copy(w_hbm.at[pl.ds(r, BLK)], w2.at[ib], sem).wait()

        def bias_and_gather(ib, row, rb):
            for i in range(CHUNK // 16):
                sl = pl.ds(i * 16, 16)
                src2[ib, row, sl] = src2[ib, row, sl] + c
            pltpu.async_copy(x_hbm.at[src2.at[ib, row]], rows.at[rb], sem_g[rb])

        def wait_gather(ib, row, rb):
            pltpu.make_async_copy(
                x_hbm.at[src2.at[ib, row]], rows.at[rb], sem_g[rb]
            ).wait()

        def issue_scatter(ib, row, rb):
            for i in range(CHUNK // 16):
                sl = pl.ds(i * 16, 16)
                dstr[rb, sl] = dst2[ib, row, sl]
            pltpu.async_copy(rows.at[rb], acc.at[dstr.at[rb]], sem_s[rb], add=True)

        def wait_scatter(rb):
            pltpu.make_async_copy(
                rows.at[rb], acc.at[dstr.at[rb]], sem_s[rb]
            ).wait()

        def scale(ib, row, rb):
            @pl.loop(0, CHUNK, step=16)
            def _(g):
                w16 = w2[ib, row, pl.ds(g, 16)]
                for kk in range(16):
                    r = g + kk
                    wr = w16[kk]
                    rows[rb, r, pl.ds(0, 16)] = rows[rb, r, pl.ds(0, 16)] * wr
                    rows[rb, r, pl.ds(16, 16)] = rows[rb, r, pl.ds(16, 16)] * wr

        pltpu.async_copy(
            z_hbm.at[pl.ds(s * STRIPE, STRIPE)],
            acc.at[pl.ds(s * STRIPE, STRIPE)],
            sem_z,
        )
        load_idx(0, 0, sem_i[0])
        wait_idx(0, 0, sem_i[0])
        load_idx(1, 1, sem_i[1])
        for kk in range(RB - 1):
            bias_and_gather(0, kk, kk)
        pltpu.make_async_copy(
            z_hbm.at[pl.ds(s * STRIPE, STRIPE)],
            acc.at[pl.ds(s * STRIPE, STRIPE)],
            sem_z,
        ).wait()
        plsc.subcore_barrier()

        @pl.loop(0, NBLK // 2)
        def _(g):
            for slot in range(2):
                ib = slot
                for kk in range(BLK):
                    wait_gather(ib, kk, kk)
                    scale(ib, kk, kk)
                    if kk == 0:
                        tib, trow = ib, 3
                    else:
                        tib, trow = 1 - ib, kk - 1
                    last_block = slot == 1 and kk >= 1
                    if kk == 1:
                        def _w(g=g, slot=slot, ib=ib):
                            wait_idx(2 * g + slot + 1, 1 - ib, sem_i[1 - ib])
                        if slot == 0:
                            _w()
                        else:
                            pl.when(g < NBLK // 2 - 1)(_w)
                    def _ws(trow=trow):
                        wait_scatter(trow)
                    if kk == 0 and slot == 0:
                        pl.when(g > 0)(_ws)
                    elif last_block:
                        pl.when(g < NBLK // 2 - 1)(_ws)
                    else:
                        _ws()
                    def _ig(tib=tib, trow=trow):
                        bias_and_gather(tib, trow, trow)
                    if last_block:
                        pl.when(g < NBLK // 2 - 1)(_ig)
                    else:
                        _ig()
                    issue_scatter(ib, kk, kk)
                    if kk == 3:
                        def _li(g=g, slot=slot, ib=ib):
                            load_idx(2 * g + slot + 2, ib, sem_i[ib])
                        pl.when(g < NBLK // 2 - 1)(_li)

        for kk in range(BLK):
            wait_scatter(kk)
        plsc.subcore_barrier()
        pltpu.sync_copy(
            acc.at[pl.ds(s * STRIPE, STRIPE)],
            y_hbm.at[pl.ds(s * STRIPE, STRIPE), c],
        )

    return k(srcp, dstp, wp, xflat, zrows)


G2 = NP // 2
_DB = G2 // 4
_DG = 4


def _dense_body(y_ref, wd_ref, b_ref, o_ref):
    h = jnp.dot(y_ref[...], wd_ref[...], preferred_element_type=jnp.float32)
    o_ref[...] = jnp.maximum(h + b_ref[...], 0.0)


def _dense_packed_tc(yp, wd, bd):
    return pl.pallas_call(
        _dense_body,
        grid=(_DG,),
        in_specs=[
            pl.BlockSpec((_DB, 2 * D), lambda i: (i, 0)),
            pl.BlockSpec((2 * D, 2 * D), lambda i: (0, 0)),
            pl.BlockSpec((1, 2 * D), lambda i: (0, 0)),
        ],
        out_specs=pl.BlockSpec((_DB, 2 * D), lambda i: (i, 0)),
        out_shape=jax.ShapeDtypeStruct((G2, 2 * D), jnp.float32),
    )(yp, wd, bd)


def _blockdiag2(w):
    z = jnp.zeros((2 * D, 2 * D), jnp.float32)
    return z.at[0:D, 0:D].set(w).at[D : 2 * D, D : 2 * D].set(w)


def kernel(edge_index, edge_weight, emb_weight, W1, b1, W2, b2):
    pad = E_PAD - E
    srcp = jnp.pad(edge_index[1] * 2, (0, pad)).reshape(E_PAD // CHUNK, CHUNK)
    dstp = jnp.pad(edge_index[0], (0, pad)).reshape(E_PAD // CHUNK, CHUNK)
    wp = jnp.pad(edge_weight, (0, pad)).reshape(E_PAD // CHUNK, CHUNK)

    xpk = jnp.pad(emb_weight.reshape(N // 2, 2 * D), ((0, (NP - N) // 2), (0, 0)))
    zrows = jnp.zeros((NP, HALF), jnp.float32)
    w1d = _blockdiag2(W1)
    w2d = _blockdiag2(W2)
    b1d = jnp.concatenate([b1, b1]).reshape(1, 2 * D)
    b2d = jnp.concatenate([b2, b2]).reshape(1, 2 * D)

    y1 = _spmm_sc(srcp, dstp, wp, xpk.reshape(NCORE * NP, HALF), zrows)
    h = _dense_packed_tc(y1.reshape(G2, 2 * D), w1d, b1d)
    y2 = _spmm_sc(srcp, dstp, wp, h.reshape(NCORE * NP, HALF), zrows)
    zp = _dense_packed_tc(y2.reshape(G2, 2 * D), w2d, b2d)
    return zp.reshape(NP, D)[0:N]
